# baseline (speedup 1.0000x reference)
import jax
import jax.numpy as jnp
from jax.experimental import pallas as pl
from jax.experimental.pallas import tpu as pltpu

OUT_F = 512
IN_F = 784
VTHR = 12500.0
PROHIB = 11250.0


def _snn_step(xc_ref, xp_ref, a_ref, w_ref, out_ref, drive_s, prev_s):
    s = pl.program_id(0)

    @pl.when(s == 0)
    def _init():
        drive_s[...] = jnp.zeros_like(drive_s)
        prev_s[...] = jnp.zeros_like(prev_s)

    ind = xc_ref[0]
    ind_col = xp_ref[0]
    av = a_ref[0]

    mem_pre = jnp.maximum(drive_s[...], 0.0)
    spike = mem_pre >= VTHR
    spike_f = spike.astype(jnp.float32)
    out_ref[0] = spike_f
    mem_post = jnp.where(spike, 0.0, mem_pre)

    delta = jnp.where(spike, av, prev_s[...] * ind)
    w_new = jnp.clip(w_ref[...] + delta, 0.0, 127.0)
    w_ref[...] = w_new

    psum = jax.lax.dot_general(
        w_new, ind_col, (((1,), (0,)), ((), ())),
        preferred_element_type=jnp.float32,
    )

    anyspk = jnp.sum(spike_f)
    prohibit = jnp.where(anyspk > 0.0, PROHIB, 0.0)
    drive_s[...] = mem_post + psum - prohibit
    s0 = jnp.where(s == 0, 1.0, 0.0)
    prev_s[...] = -jnp.maximum(spike_f, s0)


def _run(x, weight):
    T = x.shape[0]
    xf = x.reshape(T, 1, IN_F)
    xprev = jnp.concatenate([jnp.ones((1, 1, IN_F), jnp.float32), xf[:-1]], axis=0)
    a = 2.0 * xf + (1.0 - xf) * xprev
    a_in = jnp.concatenate([jnp.zeros((1, 1, IN_F), jnp.float32), a], axis=0)
    x_cur = jnp.concatenate([xf[:1], xf], axis=0)
    x_psum = jnp.concatenate([xf, xf[:1]], axis=0)
    x_psum_col = x_psum.reshape(T + 1, IN_F, 1)

    spikes = pl.pallas_call(
        _snn_step,
        grid=(T + 1,),
        in_specs=[
            pl.BlockSpec((1, 1, IN_F), lambda s: (s, 0, 0)),
            pl.BlockSpec((1, IN_F, 1), lambda s: (s, 0, 0)),
            pl.BlockSpec((1, 1, IN_F), lambda s: (s, 0, 0)),
            pl.BlockSpec((OUT_F, IN_F), lambda s: (0, 0)),
        ],
        out_specs=pl.BlockSpec((1, OUT_F, 1),
                               lambda s: (jnp.maximum(s - 1, 0), 0, 0)),
        out_shape=jax.ShapeDtypeStruct((T, OUT_F, 1), jnp.float32),
        scratch_shapes=[
            pltpu.VMEM((OUT_F, 1), jnp.float32),
            pltpu.VMEM((OUT_F, 1), jnp.float32),
        ],
        compiler_params=pltpu.CompilerParams(
            dimension_semantics=("arbitrary",),
        ),
    )(x_cur, x_psum_col, a_in, weight)
    return spikes


def kernel(x, weight):
    T = x.shape[0]
    spikes = _run(x, weight)
    return spikes.reshape(T, 1, OUT_F)


# speedup vs baseline: 1.4145x; 1.4145x over previous
import jax
import jax.numpy as jnp
from jax.experimental import pallas as pl
from jax.experimental.pallas import tpu as pltpu

OUT_F = 512
IN_F = 784
VTHR = 12500.0
PROHIB = 11250.0


def _snn_step(xc_ref, xp_ref, a_ref, w_ref, out_ref, drive_s, prev_s):
    s = pl.program_id(0)

    @pl.when(s == 0)
    def _init():
        drive_s[...] = jnp.zeros_like(drive_s)
        prev_s[...] = jnp.zeros_like(prev_s)

    ind = xc_ref[s]
    ind_next = xp_ref[s]
    av = a_ref[s]

    mem_pre = jnp.maximum(drive_s[...], 0.0)
    spike = mem_pre >= VTHR
    spike_f = spike.astype(jnp.float32)
    out_ref[0] = spike_f
    mem_post = jnp.where(spike, 0.0, mem_pre)

    delta = jnp.where(spike, av, prev_s[...] * ind)
    w_new = jnp.clip(w_ref[...] + delta, 0.0, 127.0)
    w_ref[...] = w_new

    psum = jnp.sum(w_new * ind_next, axis=1, keepdims=True)

    anyspk = jnp.sum(spike_f)
    prohibit = jnp.where(anyspk > 0.0, PROHIB, 0.0)
    drive_s[...] = mem_post + psum - prohibit
    s0 = jnp.where(s == 0, 1.0, 0.0)
    prev_s[...] = -jnp.maximum(spike_f, s0)


def _run(x, weight):
    T = x.shape[0]
    xf = x.reshape(T, 1, IN_F)
    xprev = jnp.concatenate([jnp.ones((1, 1, IN_F), jnp.float32), xf[:-1]], axis=0)
    a = 2.0 * xf + (1.0 - xf) * xprev
    a_in = jnp.concatenate([jnp.zeros((1, 1, IN_F), jnp.float32), a], axis=0)
    x_cur = jnp.concatenate([xf[:1], xf], axis=0)
    x_psum = jnp.concatenate([xf, xf[:1]], axis=0)

    spikes = pl.pallas_call(
        _snn_step,
        grid=(T + 1,),
        in_specs=[
            pl.BlockSpec((T + 1, 1, IN_F), lambda s: (0, 0, 0)),
            pl.BlockSpec((T + 1, 1, IN_F), lambda s: (0, 0, 0)),
            pl.BlockSpec((T + 1, 1, IN_F), lambda s: (0, 0, 0)),
            pl.BlockSpec((OUT_F, IN_F), lambda s: (0, 0)),
        ],
        out_specs=pl.BlockSpec((1, OUT_F, 1),
                               lambda s: (jnp.maximum(s - 1, 0), 0, 0)),
        out_shape=jax.ShapeDtypeStruct((T, OUT_F, 1), jnp.float32),
        scratch_shapes=[
            pltpu.VMEM((OUT_F, 1), jnp.float32),
            pltpu.VMEM((OUT_F, 1), jnp.float32),
        ],
        compiler_params=pltpu.CompilerParams(
            dimension_semantics=("arbitrary",),
        ),
    )(x_cur, x_psum, a_in, weight)
    return spikes


def kernel(x, weight):
    T = x.shape[0]
    spikes = _run(x, weight)
    return spikes.reshape(T, 1, OUT_F)


# speedup vs baseline: 1.4812x; 1.0472x over previous
import jax
import jax.numpy as jnp
from jax.experimental import pallas as pl
from jax.experimental.pallas import tpu as pltpu

OUT_F = 512
IN_F = 784
VTHR = 12500.0
PROHIB = 11250.0


def _snn_step(x_ref, a_ref, w_ref, out_ref, drive_s, prev_s):
    s = pl.program_id(0)
    T = x_ref.shape[0]

    @pl.when(s == 0)
    def _init():
        drive_s[...] = jnp.zeros_like(drive_s)
        prev_s[...] = jnp.zeros_like(prev_s)

    tm1 = jnp.maximum(s - 1, 0)
    ind = x_ref[tm1]
    ind_next = x_ref[jnp.minimum(s, T - 1)]
    av = a_ref[tm1]

    mem_pre = jnp.maximum(drive_s[...], 0.0)
    spike = mem_pre >= VTHR
    spike_f = spike.astype(jnp.float32)
    out_ref[0] = jnp.transpose(spike_f)
    mem_post = jnp.where(spike, 0.0, mem_pre)

    delta = jnp.where(spike, av, prev_s[...] * ind)
    w_new = jnp.clip(w_ref[...] + delta, 0.0, 127.0)
    w_ref[...] = w_new

    psum = jnp.sum(w_new * ind_next, axis=1, keepdims=True)

    anyspk = jnp.sum(spike_f)
    prohibit = jnp.where(anyspk > 0.0, PROHIB, 0.0)
    drive_s[...] = mem_post + psum - prohibit
    s0 = jnp.where(s == 0, 1.0, 0.0)
    prev_s[...] = -jnp.maximum(spike_f, s0)


def _run(x, weight):
    T = x.shape[0]
    xf = x.reshape(T, 1, IN_F)
    xprev = jnp.concatenate([jnp.ones((1, 1, IN_F), jnp.float32), xf[:-1]], axis=0)
    a = 2.0 * xf + (1.0 - xf) * xprev

    spikes = pl.pallas_call(
        _snn_step,
        grid=(T + 1,),
        in_specs=[
            pl.BlockSpec((T, 1, IN_F), lambda s: (0, 0, 0)),
            pl.BlockSpec((T, 1, IN_F), lambda s: (0, 0, 0)),
            pl.BlockSpec((OUT_F, IN_F), lambda s: (0, 0)),
        ],
        out_specs=pl.BlockSpec((1, 1, OUT_F),
                               lambda s: (jnp.maximum(s - 1, 0), 0, 0)),
        out_shape=jax.ShapeDtypeStruct((T, 1, OUT_F), jnp.float32),
        scratch_shapes=[
            pltpu.VMEM((OUT_F, 1), jnp.float32),
            pltpu.VMEM((OUT_F, 1), jnp.float32),
        ],
        compiler_params=pltpu.CompilerParams(
            dimension_semantics=("arbitrary",),
        ),
    )(xf, a, weight)
    return spikes


def kernel(x, weight):
    return _run(x, weight)


# speedup vs baseline: 1.5822x; 1.0682x over previous
import jax
import jax.numpy as jnp
from jax.experimental import pallas as pl
from jax.experimental.pallas import tpu as pltpu

OUT_F = 512
IN_F = 784
VTHR = 12500.0
PROHIB = 11250.0


def _snn_step(x_ref, a_ref, w_ref, out_ref, drive_s, prevc_s):
    s = pl.program_id(0)
    T = x_ref.shape[0]

    @pl.when(s == 0)
    def _init():
        drive_s[...] = jnp.zeros_like(drive_s)
        prevc_s[...] = jnp.zeros_like(prevc_s)

    tm1 = jnp.maximum(s - 1, 0)
    ind = x_ref[tm1]
    av = a_ref[tm1]
    ind_next = x_ref[jnp.minimum(s, T - 1)]

    mem_pre = jnp.maximum(drive_s[...], 0.0)
    spike_r = mem_pre >= VTHR
    spike_fr = spike_r.astype(jnp.float32)
    out_ref[0] = spike_fr
    mem_post = jnp.where(spike_r, 0.0, mem_pre)

    spike_fc = jnp.transpose(spike_fr)
    delta = jnp.where(spike_fc != 0.0, av, prevc_s[...] * ind)
    w_new = jnp.clip(w_ref[...] + delta, 0.0, 127.0)
    w_ref[...] = w_new

    psum_row = jax.lax.dot_general(
        ind_next, w_new, (((1,), (1,)), ((), ())),
        preferred_element_type=jnp.float32)

    anyspk = jnp.sum(spike_fr)
    prohibit = jnp.where(anyspk > 0.0, PROHIB, 0.0)
    drive_s[...] = mem_post + psum_row - prohibit
    s0 = jnp.where(s == 0, 1.0, 0.0)
    prevc_s[...] = -jnp.maximum(spike_fc, s0)


def _run(x, weight):
    T = x.shape[0]
    xf = x.reshape(T, 1, IN_F)
    xprev = jnp.concatenate([jnp.ones((1, 1, IN_F), jnp.float32), xf[:-1]], axis=0)
    a = 2.0 * xf + (1.0 - xf) * xprev

    spikes = pl.pallas_call(
        _snn_step,
        grid=(T + 1,),
        in_specs=[
            pl.BlockSpec((T, 1, IN_F), lambda s: (0, 0, 0)),
            pl.BlockSpec((T, 1, IN_F), lambda s: (0, 0, 0)),
            pl.BlockSpec((OUT_F, IN_F), lambda s: (0, 0)),
        ],
        out_specs=pl.BlockSpec((1, 1, OUT_F),
                               lambda s: (jnp.maximum(s - 1, 0), 0, 0)),
        out_shape=jax.ShapeDtypeStruct((T, 1, OUT_F), jnp.float32),
        scratch_shapes=[
            pltpu.VMEM((1, OUT_F), jnp.float32),
            pltpu.VMEM((OUT_F, 1), jnp.float32),
        ],
        compiler_params=pltpu.CompilerParams(
            dimension_semantics=("arbitrary",),
        ),
    )(xf, a, weight)
    return spikes


def kernel(x, weight):
    return _run(x, weight)


# speedup vs baseline: 1.5920x; 1.0062x over previous
import jax
import jax.numpy as jnp
from jax.experimental import pallas as pl
from jax.experimental.pallas import tpu as pltpu

OUT_F = 512
IN_F = 784
VTHR = 12500.0
PROHIB = 11250.0


def _one_step(u, is_warm, x_ref, a_ref, w_ref, out_ref, drive_s, prevc_s):
    T = x_ref.shape[0]
    tm1 = jnp.clip(u - 1, 0, T - 1)
    ind = x_ref[tm1]
    av = a_ref[tm1]
    ind_next = x_ref[jnp.minimum(u, T - 1)]

    mem_pre = jnp.maximum(drive_s[...], 0.0)
    spike_r = mem_pre >= VTHR
    spike_fr = spike_r.astype(jnp.float32)
    out_ref[jnp.minimum(u, T + 1)] = spike_fr
    mem_post = jnp.where(spike_r, 0.0, mem_pre)

    spike_fc = jnp.transpose(spike_fr)
    delta = jnp.where(spike_fc != 0.0, av, prevc_s[...] * ind)
    w_new = jnp.clip(w_ref[...] + delta, 0.0, 127.0)
    w_ref[...] = w_new

    psum_row = jax.lax.dot_general(
        ind_next, w_new, (((1,), (1,)), ((), ())),
        preferred_element_type=jnp.float32)

    anyspk = jnp.sum(spike_fr)
    prohibit = jnp.where(anyspk > 0.0, PROHIB, 0.0)
    drive_s[...] = mem_post + psum_row - prohibit
    if is_warm:
        s0 = jnp.where(u == 0, 1.0, 0.0)
        prevc_s[...] = -jnp.maximum(spike_fc, s0)
    else:
        prevc_s[...] = -spike_fc


def _snn_pair(x_ref, a_ref, w_ref, out_ref, drive_s, prevc_s):
    s = pl.program_id(0)

    @pl.when(s == 0)
    def _init():
        drive_s[...] = jnp.zeros_like(drive_s)
        prevc_s[...] = jnp.zeros_like(prevc_s)

    _one_step(2 * s, True, x_ref, a_ref, w_ref, out_ref, drive_s, prevc_s)
    _one_step(2 * s + 1, False, x_ref, a_ref, w_ref, out_ref, drive_s, prevc_s)


def _run(x, weight):
    T = x.shape[0]
    xf = x.reshape(T, 1, IN_F)
    xprev = jnp.concatenate([jnp.ones((1, 1, IN_F), jnp.float32), xf[:-1]], axis=0)
    a = 2.0 * xf + (1.0 - xf) * xprev

    spikes_full = pl.pallas_call(
        _snn_pair,
        grid=((T + 2) // 2,),
        in_specs=[
            pl.BlockSpec((T, 1, IN_F), lambda s: (0, 0, 0)),
            pl.BlockSpec((T, 1, IN_F), lambda s: (0, 0, 0)),
            pl.BlockSpec((OUT_F, IN_F), lambda s: (0, 0)),
        ],
        out_specs=pl.BlockSpec((T + 2, 1, OUT_F), lambda s: (0, 0, 0)),
        out_shape=jax.ShapeDtypeStruct((T + 2, 1, OUT_F), jnp.float32),
        scratch_shapes=[
            pltpu.VMEM((1, OUT_F), jnp.float32),
            pltpu.VMEM((OUT_F, 1), jnp.float32),
        ],
        compiler_params=pltpu.CompilerParams(
            dimension_semantics=("arbitrary",),
        ),
    )(xf, a, weight)
    return spikes_full[1:T + 1]


def kernel(x, weight):
    return _run(x, weight)


# speedup vs baseline: 1.5951x; 1.0019x over previous
import jax
import jax.numpy as jnp
from jax.experimental import pallas as pl
from jax.experimental.pallas import tpu as pltpu

OUT_F = 512
IN_F = 784
VTHR = 12500.0
PROHIB = 11250.0


def _one_step(u, slot, is_warm, x_ref, a_ref, w_ref, out_ref, drive_s, prevc_s):
    T = x_ref.shape[0]
    tm1 = jnp.clip(u - 1, 0, T - 1)
    ind = x_ref[tm1]
    av = a_ref[tm1]
    ind_next = x_ref[jnp.minimum(u, T - 1)]

    mem_pre = jnp.maximum(drive_s[...], 0.0)
    spike_r = mem_pre >= VTHR
    spike_fr = spike_r.astype(jnp.float32)
    out_ref[slot] = spike_fr
    mem_post = jnp.where(spike_r, 0.0, mem_pre)

    spike_fc = jnp.transpose(spike_fr)
    delta = jnp.where(spike_fc != 0.0, av, prevc_s[...] * ind)
    w_new = jnp.clip(w_ref[...] + delta, 0.0, 127.0)
    w_ref[...] = w_new

    psum_row = jax.lax.dot_general(
        ind_next, w_new, (((1,), (1,)), ((), ())),
        preferred_element_type=jnp.float32)

    anyspk = jnp.sum(spike_fr)
    prohibit = jnp.where(anyspk > 0.0, PROHIB, 0.0)
    drive_s[...] = mem_post + psum_row - prohibit
    if is_warm:
        s0 = jnp.where(u == 0, 1.0, 0.0)
        prevc_s[...] = -jnp.maximum(spike_fc, s0)
    else:
        prevc_s[...] = -spike_fc


def _snn_pair(x_ref, a_ref, w_ref, out_ref, drive_s, prevc_s):
    s = pl.program_id(0)

    @pl.when(s == 0)
    def _init():
        drive_s[...] = jnp.zeros_like(drive_s)
        prevc_s[...] = jnp.zeros_like(prevc_s)

    _one_step(2 * s, 0, True, x_ref, a_ref, w_ref, out_ref, drive_s, prevc_s)
    _one_step(2 * s + 1, 1, False, x_ref, a_ref, w_ref, out_ref, drive_s, prevc_s)


def _run(x, weight):
    T = x.shape[0]
    xf = x.reshape(T, 1, IN_F)
    xprev = jnp.concatenate([jnp.ones((1, 1, IN_F), jnp.float32), xf[:-1]], axis=0)
    a = 2.0 * xf + (1.0 - xf) * xprev

    spikes_full = pl.pallas_call(
        _snn_pair,
        grid=((T + 2) // 2,),
        in_specs=[
            pl.BlockSpec((T, 1, IN_F), lambda s: (0, 0, 0)),
            pl.BlockSpec((T, 1, IN_F), lambda s: (0, 0, 0)),
            pl.BlockSpec((OUT_F, IN_F), lambda s: (0, 0)),
        ],
        out_specs=pl.BlockSpec((2, 1, OUT_F), lambda s: (s, 0, 0)),
        out_shape=jax.ShapeDtypeStruct((T + 2, 1, OUT_F), jnp.float32),
        scratch_shapes=[
            pltpu.VMEM((1, OUT_F), jnp.float32),
            pltpu.VMEM((OUT_F, 1), jnp.float32),
        ],
        compiler_params=pltpu.CompilerParams(
            dimension_semantics=("arbitrary",),
        ),
    )(xf, a, weight)
    return spikes_full[1:T + 1]


def kernel(x, weight):
    return _run(x, weight)


# speedup vs baseline: 1.6822x; 1.0546x over previous
import jax
import jax.numpy as jnp
from jax.experimental import pallas as pl
from jax.experimental.pallas import tpu as pltpu

OUT_F = 512
IN_F = 784
VTHR = 12500.0
PROHIB = 11250.0


def _one_step(u, slot, is_warm, x_ref, a_ref, eye_ref, w_ref, out_ref, drive_s, prevc_s):
    T = x_ref.shape[0]
    tm1 = jnp.clip(u - 1, 0, T - 1)
    ind = x_ref[tm1]
    av = a_ref[tm1]
    ind_next = x_ref[jnp.minimum(u, T - 1)]

    mem_pre = jnp.maximum(drive_s[...], 0.0)
    spike_r = mem_pre >= VTHR
    spike_fr = spike_r.astype(jnp.float32)
    out_ref[slot] = spike_fr
    mem_post = jnp.where(spike_r, 0.0, mem_pre)

    spike_fc = jax.lax.dot_general(
        eye_ref[...], spike_fr, (((1,), (1,)), ((), ())),
        preferred_element_type=jnp.float32)
    delta = jnp.where(spike_fc != 0.0, av, prevc_s[...] * ind)
    w_new = jnp.clip(w_ref[...] + delta, 0.0, 127.0)
    w_ref[...] = w_new

    psum_row = jax.lax.dot_general(
        ind_next, w_new, (((1,), (1,)), ((), ())),
        preferred_element_type=jnp.float32)

    anyspk = jnp.sum(spike_fr)
    prohibit = jnp.where(anyspk > 0.0, PROHIB, 0.0)
    drive_s[...] = mem_post + psum_row - prohibit
    if is_warm:
        s0 = jnp.where(u == 0, 1.0, 0.0)
        prevc_s[...] = -jnp.maximum(spike_fc, s0)
    else:
        prevc_s[...] = -spike_fc


def _snn_pair(x_ref, a_ref, eye_ref, w_ref, out_ref, drive_s, prevc_s):
    s = pl.program_id(0)

    @pl.when(s == 0)
    def _init():
        drive_s[...] = jnp.zeros_like(drive_s)
        prevc_s[...] = jnp.zeros_like(prevc_s)

    _one_step(2 * s, 0, True, x_ref, a_ref, eye_ref, w_ref, out_ref, drive_s, prevc_s)
    _one_step(2 * s + 1, 1, False, x_ref, a_ref, eye_ref, w_ref, out_ref, drive_s, prevc_s)


def _run(x, weight):
    T = x.shape[0]
    xf = x.reshape(T, 1, IN_F)
    xprev = jnp.concatenate([jnp.ones((1, 1, IN_F), jnp.float32), xf[:-1]], axis=0)
    a = 2.0 * xf + (1.0 - xf) * xprev

    spikes_full = pl.pallas_call(
        _snn_pair,
        grid=((T + 2) // 2,),
        in_specs=[
            pl.BlockSpec((T, 1, IN_F), lambda s: (0, 0, 0)),
            pl.BlockSpec((T, 1, IN_F), lambda s: (0, 0, 0)),
            pl.BlockSpec((OUT_F, OUT_F), lambda s: (0, 0)),
            pl.BlockSpec((OUT_F, IN_F), lambda s: (0, 0)),
        ],
        out_specs=pl.BlockSpec((2, 1, OUT_F), lambda s: (s, 0, 0)),
        out_shape=jax.ShapeDtypeStruct((T + 2, 1, OUT_F), jnp.float32),
        scratch_shapes=[
            pltpu.VMEM((1, OUT_F), jnp.float32),
            pltpu.VMEM((OUT_F, 1), jnp.float32),
        ],
        compiler_params=pltpu.CompilerParams(
            dimension_semantics=("arbitrary",),
        ),
    )(xf, a, jnp.eye(OUT_F, dtype=jnp.float32), weight)
    return spikes_full[1:T + 1]


def kernel(x, weight):
    return _run(x, weight)


# speedup vs baseline: 1.6823x; 1.0001x over previous
import jax
import jax.numpy as jnp
from jax.experimental import pallas as pl
from jax.experimental.pallas import tpu as pltpu

OUT_F = 512
IN_F = 784
VTHR = 12500.0
PROHIB = 11250.0
STEPS_PER_ITER = 2


def _one_step(u, slot, is_warm, x_ref, a_ref, eye_ref, w_ref, drive_ref,
              prevc_ref, out_ref):
    T = x_ref.shape[0]
    tm1 = jnp.clip(u - 1, 0, T - 1)
    ind = x_ref[tm1]
    av = a_ref[tm1]
    ind_next = x_ref[jnp.minimum(u, T - 1)]

    mem_pre = jnp.maximum(drive_ref[...], 0.0)
    spike_r = mem_pre >= VTHR
    spike_fr = spike_r.astype(jnp.float32)
    out_ref[slot] = spike_fr
    mem_post = jnp.where(spike_r, 0.0, mem_pre)

    spike_fc = jax.lax.dot_general(
        eye_ref[...], spike_fr, (((1,), (1,)), ((), ())),
        preferred_element_type=jnp.float32)
    delta = jnp.where(spike_fc != 0.0, av, prevc_ref[...] * ind)
    w_new = jnp.clip(w_ref[...] + delta, 0.0, 127.0)
    w_ref[...] = w_new

    psum_row = jax.lax.dot_general(
        ind_next, w_new, (((1,), (1,)), ((), ())),
        preferred_element_type=jnp.float32)

    anyspk = jnp.sum(spike_fr)
    prohibit = jnp.where(anyspk > 0.0, PROHIB, 0.0)
    drive_ref[...] = mem_post + psum_row - prohibit
    if is_warm:
        s0 = jnp.where(u == 0, 1.0, 0.0)
        prevc_ref[...] = -jnp.maximum(spike_fc, s0)
    else:
        prevc_ref[...] = -spike_fc


def _snn_iter(x_ref, a_ref, eye_ref, w_ref, drive_ref, prevc_ref, out_ref):
    s = pl.program_id(0)
    for k in range(STEPS_PER_ITER):
        _one_step(STEPS_PER_ITER * s + k, k, k == 0, x_ref, a_ref, eye_ref,
                  w_ref, drive_ref, prevc_ref, out_ref)


def _run(x, weight):
    T = x.shape[0]
    xf = x.reshape(T, 1, IN_F)
    xprev = jnp.concatenate([jnp.ones((1, 1, IN_F), jnp.float32), xf[:-1]], axis=0)
    a = 2.0 * xf + (1.0 - xf) * xprev

    n_iter = (T + STEPS_PER_ITER) // STEPS_PER_ITER
    n_rows = n_iter * STEPS_PER_ITER
    full = pl.BlockSpec
    spikes_full = pl.pallas_call(
        _snn_iter,
        grid=(n_iter,),
        in_specs=[
            full((T, 1, IN_F), lambda s: (0, 0, 0)),
            full((T, 1, IN_F), lambda s: (0, 0, 0)),
            full((OUT_F, OUT_F), lambda s: (0, 0)),
            full((OUT_F, IN_F), lambda s: (0, 0)),
            full((1, OUT_F), lambda s: (0, 0)),
            full((OUT_F, 1), lambda s: (0, 0)),
        ],
        out_specs=pl.BlockSpec((STEPS_PER_ITER, 1, OUT_F), lambda s: (s, 0, 0)),
        out_shape=jax.ShapeDtypeStruct((n_rows, 1, OUT_F), jnp.float32),
        compiler_params=pltpu.CompilerParams(
            dimension_semantics=("arbitrary",),
        ),
    )(xf, a, jnp.eye(OUT_F, dtype=jnp.float32), weight,
      jnp.zeros((1, OUT_F), jnp.float32), jnp.zeros((OUT_F, 1), jnp.float32))
    return spikes_full[1:T + 1]


def kernel(x, weight):
    return _run(x, weight)
